# Initial kernel scaffold; baseline (speedup 1.0000x reference)
import jax
import jax.numpy as jnp
from jax.experimental import pallas as pl


def kernel(uvm_gt, uvm_out, triangles):
    raise NotImplementedError("write your pallas kernel here")


# speedup vs baseline: 4.5426x; 4.5426x over previous
import functools

import jax
import jax.numpy as jnp
import numpy as np
from jax import lax
from jax.experimental import pallas as pl
from jax.experimental.pallas import tpu as pltpu
from jax.experimental.pallas import tpu_sc as plsc

NC = 2
NS = 16
L = 16

B, C, H, W = 16, 3, 256, 256
T = 16384
NW = NC * NS
TPW = T // NW
CH = 128
NCHUNK = TPW // CH
EPS2 = np.float32(1e-24)
INV_EPS = np.float32(1e12)
MEAN_SCALE = np.float32(1.0 / (B * 3 * T))


def _rsqrt(s):
    i = lax.bitcast_convert_type(s, jnp.int32)
    i = np.int32(0x5F3759DF) - (i >> 1)
    r = lax.bitcast_convert_type(i, jnp.float32)
    hs = s * np.float32(-0.5)
    r = r * (np.float32(1.5) + hs * r * r)
    r = r * (np.float32(1.5) + hs * r * r)
    return jnp.where(s < EPS2, INV_EPS, r)


def _sc_body(tab_o, tab_g, tri, out,
             tri_v, idx0, idx1, idx2,
             ro0, ro1, ro2, rg0, rg1, rg2,
             accv, shv, redv, sem):
    cid = lax.axis_index("c")
    sid = lax.axis_index("s")
    wid = sid * NC + cid

    acc = jnp.zeros((L,), jnp.float32)
    for chunk in range(NCHUNK):
        base = wid * TPW + chunk * CH
        pltpu.sync_copy(tri.at[:, pl.ds(base, CH)], tri_v)
        for g in range(CH // L):
            sl = pl.ds(g * L, L)
            idx0[sl] = (tri_v[0, sl] << 8) + tri_v[1, sl]
            idx1[sl] = (tri_v[2, sl] << 8) + tri_v[3, sl]
            idx2[sl] = (tri_v[4, sl] << 8) + tri_v[5, sl]
        cps = [
            pltpu.async_copy(tab_o.at[idx0], ro0, sem),
            pltpu.async_copy(tab_o.at[idx1], ro1, sem),
            pltpu.async_copy(tab_o.at[idx2], ro2, sem),
            pltpu.async_copy(tab_g.at[idx0], rg0, sem),
            pltpu.async_copy(tab_g.at[idx1], rg1, sem),
            pltpu.async_copy(tab_g.at[idx2], rg2, sem),
        ]
        for cp in cps:
            cp.wait()

        def tri_body(i, a):
            e1x = ro1[i, pl.ds(0 * L, L)] - ro0[i, pl.ds(0 * L, L)]
            e1y = ro1[i, pl.ds(1 * L, L)] - ro0[i, pl.ds(1 * L, L)]
            e1z = ro1[i, pl.ds(2 * L, L)] - ro0[i, pl.ds(2 * L, L)]
            e2x = ro2[i, pl.ds(0 * L, L)] - ro0[i, pl.ds(0 * L, L)]
            e2y = ro2[i, pl.ds(1 * L, L)] - ro0[i, pl.ds(1 * L, L)]
            e2z = ro2[i, pl.ds(2 * L, L)] - ro0[i, pl.ds(2 * L, L)]
            e3x = ro2[i, pl.ds(0 * L, L)] - ro1[i, pl.ds(0 * L, L)]
            e3y = ro2[i, pl.ds(1 * L, L)] - ro1[i, pl.ds(1 * L, L)]
            e3z = ro2[i, pl.ds(2 * L, L)] - ro1[i, pl.ds(2 * L, L)]
            f1x = rg1[i, pl.ds(0 * L, L)] - rg0[i, pl.ds(0 * L, L)]
            f1y = rg1[i, pl.ds(1 * L, L)] - rg0[i, pl.ds(1 * L, L)]
            f1z = rg1[i, pl.ds(2 * L, L)] - rg0[i, pl.ds(2 * L, L)]
            f2x = rg2[i, pl.ds(0 * L, L)] - rg0[i, pl.ds(0 * L, L)]
            f2y = rg2[i, pl.ds(1 * L, L)] - rg0[i, pl.ds(1 * L, L)]
            f2z = rg2[i, pl.ds(2 * L, L)] - rg0[i, pl.ds(2 * L, L)]
            nx = f1y * f2z - f1z * f2y
            ny = f1z * f2x - f1x * f2z
            nz = f1x * f2y - f1y * f2x
            rn = _rsqrt(nx * nx + ny * ny + nz * nz)
            r1 = _rsqrt(e1x * e1x + e1y * e1y + e1z * e1z)
            r2 = _rsqrt(e2x * e2x + e2y * e2y + e2z * e2z)
            r3 = _rsqrt(e3x * e3x + e3y * e3y + e3z * e3z)
            c1 = jnp.abs(e1x * nx + e1y * ny + e1z * nz) * r1
            c2 = jnp.abs(e2x * nx + e2y * ny + e2z * nz) * r2
            c3 = jnp.abs(e3x * nx + e3y * ny + e3z * nz) * r3
            return a + (c1 + c2 + c3) * rn

        acc = lax.fori_loop(0, CH, tri_body, acc)

    accv[...] = acc * MEAN_SCALE
    pltpu.sync_copy(accv, shv.at[sid])
    plsc.subcore_barrier()

    @pl.when(sid == 0)
    def _():
        pltpu.sync_copy(shv, redv)
        tot = redv[0]
        for s in range(1, NS):
            tot = tot + redv[s]
        accv[...] = tot
        pltpu.sync_copy(accv, out.at[cid])


_sc_loss = functools.partial(
    pl.kernel,
    out_type=jax.ShapeDtypeStruct((NC, L), jnp.float32),
    mesh=plsc.VectorSubcoreMesh(
        core_axis_name="c", subcore_axis_name="s",
        num_cores=NC, num_subcores=NS),
    compiler_params=pltpu.CompilerParams(use_tc_tiling_on_sc=False),
    scratch_types=[
        pltpu.VMEM((6, CH), jnp.int32),
        pltpu.VMEM((CH,), jnp.int32),
        pltpu.VMEM((CH,), jnp.int32),
        pltpu.VMEM((CH,), jnp.int32),
        pltpu.VMEM((CH, C * L), jnp.float32),
        pltpu.VMEM((CH, C * L), jnp.float32),
        pltpu.VMEM((CH, C * L), jnp.float32),
        pltpu.VMEM((CH, C * L), jnp.float32),
        pltpu.VMEM((CH, C * L), jnp.float32),
        pltpu.VMEM((CH, C * L), jnp.float32),
        pltpu.VMEM((L,), jnp.float32),
        pltpu.VMEM_SHARED((NS, L), jnp.float32),
        pltpu.VMEM((NS, L), jnp.float32),
        pltpu.SemaphoreType.DMA,
    ],
)(_sc_body)


def kernel(uvm_gt, uvm_out, triangles):
    tab_g = jnp.transpose(uvm_gt, (2, 3, 1, 0)).reshape(H * W, C * B)
    tab_o = jnp.transpose(uvm_out, (2, 3, 1, 0)).reshape(H * W, C * B)
    tri_t = triangles.T
    part = _sc_loss(tab_o, tab_g, tri_t)
    return part.sum()


# speedup vs baseline: 4.8617x; 1.0702x over previous
import functools

import jax
import jax.numpy as jnp
import numpy as np
from jax import lax
from jax.experimental import pallas as pl
from jax.experimental.pallas import tpu as pltpu
from jax.experimental.pallas import tpu_sc as plsc

NC = 2
NS = 16
L = 16

B, C, H, W = 16, 3, 256, 256
T = 16384
D = 128
NW = NC * NS
TPW = T // NW
CH = 128
NCHUNK = TPW // CH
EPS2 = np.float32(1e-24)
INV_EPS = np.float32(1e12)
MEAN_SCALE = np.float32(1.0 / (B * 3 * T))


def _rsqrt(s):
    i = lax.bitcast_convert_type(s, jnp.int32)
    i = np.int32(0x5F3759DF) - (i >> 1)
    r = lax.bitcast_convert_type(i, jnp.float32)
    hs = s * np.float32(-0.5)
    r = r * (np.float32(1.5) + hs * r * r)
    r = r * (np.float32(1.5) + hs * r * r)
    return jnp.where(s < EPS2, INV_EPS, r)


def _sc_body(tab, tri, out,
             tri_v, idx0, idx1, idx2,
             r0, r1, r2,
             accv, outv, shv, redv, sem):
    cid = lax.axis_index("c")
    sid = lax.axis_index("s")
    wid = sid * NC + cid

    iota6 = lax.iota(jnp.int32, L) * 6
    acc = jnp.zeros((L,), jnp.float32)
    for chunk in range(NCHUNK):
        base = wid * TPW + chunk * CH
        pltpu.sync_copy(tri.at[pl.ds(base * 6, CH * 6)], tri_v)
        for g in range(CH // L):
            b6 = g * (L * 6)
            t0 = plsc.load_gather(tri_v, [iota6 + b6])
            t1 = plsc.load_gather(tri_v, [iota6 + (b6 + 1)])
            t2 = plsc.load_gather(tri_v, [iota6 + (b6 + 2)])
            t3 = plsc.load_gather(tri_v, [iota6 + (b6 + 3)])
            t4 = plsc.load_gather(tri_v, [iota6 + (b6 + 4)])
            t5 = plsc.load_gather(tri_v, [iota6 + (b6 + 5)])
            sl = pl.ds(g * L, L)
            idx0[sl] = (t0 << 8) + t1
            idx1[sl] = (t2 << 8) + t3
            idx2[sl] = (t4 << 8) + t5
        cps = [
            pltpu.async_copy(tab.at[idx0], r0, sem),
            pltpu.async_copy(tab.at[idx1], r1, sem),
            pltpu.async_copy(tab.at[idx2], r2, sem),
        ]
        for cp in cps:
            cp.wait()

        def tri_body(i, a):
            e1x = r1[i, pl.ds(0, L)] - r0[i, pl.ds(0, L)]
            e1y = r1[i, pl.ds(16, L)] - r0[i, pl.ds(16, L)]
            e1z = r1[i, pl.ds(32, L)] - r0[i, pl.ds(32, L)]
            e2x = r2[i, pl.ds(0, L)] - r0[i, pl.ds(0, L)]
            e2y = r2[i, pl.ds(16, L)] - r0[i, pl.ds(16, L)]
            e2z = r2[i, pl.ds(32, L)] - r0[i, pl.ds(32, L)]
            e3x = r2[i, pl.ds(0, L)] - r1[i, pl.ds(0, L)]
            e3y = r2[i, pl.ds(16, L)] - r1[i, pl.ds(16, L)]
            e3z = r2[i, pl.ds(32, L)] - r1[i, pl.ds(32, L)]
            f1x = r1[i, pl.ds(48, L)] - r0[i, pl.ds(48, L)]
            f1y = r1[i, pl.ds(64, L)] - r0[i, pl.ds(64, L)]
            f1z = r1[i, pl.ds(80, L)] - r0[i, pl.ds(80, L)]
            f2x = r2[i, pl.ds(48, L)] - r0[i, pl.ds(48, L)]
            f2y = r2[i, pl.ds(64, L)] - r0[i, pl.ds(64, L)]
            f2z = r2[i, pl.ds(80, L)] - r0[i, pl.ds(80, L)]
            nx = f1y * f2z - f1z * f2y
            ny = f1z * f2x - f1x * f2z
            nz = f1x * f2y - f1y * f2x
            rn = _rsqrt(nx * nx + ny * ny + nz * nz)
            rr1 = _rsqrt(e1x * e1x + e1y * e1y + e1z * e1z)
            rr2 = _rsqrt(e2x * e2x + e2y * e2y + e2z * e2z)
            rr3 = _rsqrt(e3x * e3x + e3y * e3y + e3z * e3z)
            c1 = jnp.abs(e1x * nx + e1y * ny + e1z * nz) * rr1
            c2 = jnp.abs(e2x * nx + e2y * ny + e2z * nz) * rr2
            c3 = jnp.abs(e3x * nx + e3y * ny + e3z * nz) * rr3
            return a + (c1 + c2 + c3) * rn

        acc = lax.fori_loop(0, CH, tri_body, acc)

    accv[...] = acc * MEAN_SCALE
    pltpu.sync_copy(accv, shv.at[sid])
    plsc.subcore_barrier()

    @pl.when(sid == 0)
    def _():
        pltpu.sync_copy(shv, redv)
        tot = redv[0]
        for s in range(1, NS):
            tot = tot + redv[s]
        for k in range(D // L):
            outv[pl.ds(k * L, L)] = jnp.zeros((L,), jnp.float32)
        outv[pl.ds(0, L)] = tot
        pltpu.sync_copy(outv, out.at[cid])


_sc_loss = functools.partial(
    pl.kernel,
    out_type=jax.ShapeDtypeStruct((NC, D), jnp.float32),
    mesh=plsc.VectorSubcoreMesh(
        core_axis_name="c", subcore_axis_name="s",
        num_cores=NC, num_subcores=NS),
    compiler_params=pltpu.CompilerParams(
        use_tc_tiling_on_sc=False, needs_layout_passes=False),
    scratch_types=[
        pltpu.VMEM((CH * 6,), jnp.int32),
        pltpu.VMEM((CH,), jnp.int32),
        pltpu.VMEM((CH,), jnp.int32),
        pltpu.VMEM((CH,), jnp.int32),
        pltpu.VMEM((CH, D), jnp.float32),
        pltpu.VMEM((CH, D), jnp.float32),
        pltpu.VMEM((CH, D), jnp.float32),
        pltpu.VMEM((L,), jnp.float32),
        pltpu.VMEM((D,), jnp.float32),
        pltpu.VMEM_SHARED((NS, L), jnp.float32),
        pltpu.VMEM((NS, L), jnp.float32),
        pltpu.SemaphoreType.DMA,
    ],
)(_sc_body)


def kernel(uvm_gt, uvm_out, triangles):
    tab_o = jnp.transpose(uvm_out, (2, 3, 1, 0)).reshape(H * W, C * B)
    tab_g = jnp.transpose(uvm_gt, (2, 3, 1, 0)).reshape(H * W, C * B)
    tab = jnp.concatenate(
        [tab_o, tab_g, jnp.zeros((H * W, D - 2 * C * B), jnp.float32)], axis=1)
    part = _sc_loss(tab, triangles.reshape(-1))
    return part.sum()


# speedup vs baseline: 9.2452x; 1.9016x over previous
import functools

import jax
import jax.numpy as jnp
import numpy as np
from jax import lax
from jax.experimental import pallas as pl
from jax.experimental.pallas import tpu as pltpu
from jax.experimental.pallas import tpu_sc as plsc

NC = 2
NS = 16
L = 16

B, C, H, W = 16, 3, 256, 256
T = 16384
D = 128
NW = NC * NS
TPW = T // NW
CH = 128
NCHUNK = TPW // CH
EPS2 = np.float32(1e-24)
INV_EPS = np.float32(1e12)
MEAN_SCALE = np.float32(1.0 / (B * 3 * T))


def _rsqrt(s):
    i = lax.bitcast_convert_type(s, jnp.int32)
    i = np.int32(0x5F3759DF) - (i >> 1)
    r = lax.bitcast_convert_type(i, jnp.float32)
    hs = s * np.float32(-0.5)
    r = r * (np.float32(1.5) + hs * r * r)
    r = r * (np.float32(1.5) + hs * r * r)
    return jnp.where(s < EPS2, INV_EPS, r)


_HB = 8


def _tc_prep_body(uo_ref, ug_ref, tab_ref):
    z = jnp.zeros((W, D - 2 * C * B), jnp.float32)
    for k in range(_HB):
        o = uo_ref[:, :, k, :].reshape(C * B, W)
        g = ug_ref[:, :, k, :].reshape(C * B, W)
        tab_ref[pl.ds(k * W, W), :] = jnp.concatenate([o.T, g.T, z], axis=1)


_tc_prep = pl.pallas_call(
    _tc_prep_body,
    grid=(H // _HB,),
    in_specs=[
        pl.BlockSpec((B, C, _HB, W), lambda h: (0, 0, h, 0)),
        pl.BlockSpec((B, C, _HB, W), lambda h: (0, 0, h, 0)),
    ],
    out_specs=pl.BlockSpec((_HB * W, D), lambda h: (h, 0)),
    out_shape=jax.ShapeDtypeStruct((H * W, D), jnp.float32),
)


def _sc_body(tab, tri, out,
             tri_v, idx0, idx1, idx2,
             r0, r1, r2,
             accv, outv, shv, redv, sem):
    cid = lax.axis_index("c")
    sid = lax.axis_index("s")
    wid = sid * NC + cid

    acc = jnp.zeros((L,), jnp.float32)
    for chunk in range(NCHUNK):
        base = wid * TPW + chunk * CH
        pltpu.sync_copy(tri.at[:, pl.ds(base, CH)], tri_v)
        for g in range(CH // L):
            sl = pl.ds(g * L, L)
            idx0[sl] = (tri_v[0, sl] << 8) + tri_v[1, sl]
            idx1[sl] = (tri_v[2, sl] << 8) + tri_v[3, sl]
            idx2[sl] = (tri_v[4, sl] << 8) + tri_v[5, sl]
        cps = [
            pltpu.async_copy(tab.at[idx0], r0, sem),
            pltpu.async_copy(tab.at[idx1], r1, sem),
            pltpu.async_copy(tab.at[idx2], r2, sem),
        ]
        for cp in cps:
            cp.wait()

        def tri_body(i, a):
            e1x = r1[i, pl.ds(0, L)] - r0[i, pl.ds(0, L)]
            e1y = r1[i, pl.ds(16, L)] - r0[i, pl.ds(16, L)]
            e1z = r1[i, pl.ds(32, L)] - r0[i, pl.ds(32, L)]
            e2x = r2[i, pl.ds(0, L)] - r0[i, pl.ds(0, L)]
            e2y = r2[i, pl.ds(16, L)] - r0[i, pl.ds(16, L)]
            e2z = r2[i, pl.ds(32, L)] - r0[i, pl.ds(32, L)]
            e3x = r2[i, pl.ds(0, L)] - r1[i, pl.ds(0, L)]
            e3y = r2[i, pl.ds(16, L)] - r1[i, pl.ds(16, L)]
            e3z = r2[i, pl.ds(32, L)] - r1[i, pl.ds(32, L)]
            f1x = r1[i, pl.ds(48, L)] - r0[i, pl.ds(48, L)]
            f1y = r1[i, pl.ds(64, L)] - r0[i, pl.ds(64, L)]
            f1z = r1[i, pl.ds(80, L)] - r0[i, pl.ds(80, L)]
            f2x = r2[i, pl.ds(48, L)] - r0[i, pl.ds(48, L)]
            f2y = r2[i, pl.ds(64, L)] - r0[i, pl.ds(64, L)]
            f2z = r2[i, pl.ds(80, L)] - r0[i, pl.ds(80, L)]
            nx = f1y * f2z - f1z * f2y
            ny = f1z * f2x - f1x * f2z
            nz = f1x * f2y - f1y * f2x
            rn = _rsqrt(nx * nx + ny * ny + nz * nz)
            rr1 = _rsqrt(e1x * e1x + e1y * e1y + e1z * e1z)
            rr2 = _rsqrt(e2x * e2x + e2y * e2y + e2z * e2z)
            rr3 = _rsqrt(e3x * e3x + e3y * e3y + e3z * e3z)
            c1 = jnp.abs(e1x * nx + e1y * ny + e1z * nz) * rr1
            c2 = jnp.abs(e2x * nx + e2y * ny + e2z * nz) * rr2
            c3 = jnp.abs(e3x * nx + e3y * ny + e3z * nz) * rr3
            return a + (c1 + c2 + c3) * rn

        acc = lax.fori_loop(0, CH, tri_body, acc)

    accv[...] = acc * MEAN_SCALE
    pltpu.sync_copy(accv, shv.at[sid])
    plsc.subcore_barrier()

    @pl.when(sid == 0)
    def _():
        pltpu.sync_copy(shv, redv)
        tot = redv[0]
        for s in range(1, NS):
            tot = tot + redv[s]
        for k in range(D // L):
            outv[pl.ds(k * L, L)] = jnp.zeros((L,), jnp.float32)
        outv[pl.ds(0, L)] = tot
        pltpu.sync_copy(outv, out.at[cid])


_sc_loss = functools.partial(
    pl.kernel,
    out_type=jax.ShapeDtypeStruct((NC, D), jnp.float32),
    mesh=plsc.VectorSubcoreMesh(
        core_axis_name="c", subcore_axis_name="s",
        num_cores=NC, num_subcores=NS),
    compiler_params=pltpu.CompilerParams(
        use_tc_tiling_on_sc=False, needs_layout_passes=False),
    scratch_types=[
        pltpu.VMEM((6, CH), jnp.int32),
        pltpu.VMEM((CH,), jnp.int32),
        pltpu.VMEM((CH,), jnp.int32),
        pltpu.VMEM((CH,), jnp.int32),
        pltpu.VMEM((CH, D), jnp.float32),
        pltpu.VMEM((CH, D), jnp.float32),
        pltpu.VMEM((CH, D), jnp.float32),
        pltpu.VMEM((L,), jnp.float32),
        pltpu.VMEM((D,), jnp.float32),
        pltpu.VMEM_SHARED((NS, L), jnp.float32),
        pltpu.VMEM((NS, L), jnp.float32),
        pltpu.SemaphoreType.DMA,
    ],
)(_sc_body)


def kernel(uvm_gt, uvm_out, triangles):
    tab = _tc_prep(uvm_out, uvm_gt)
    part = _sc_loss(tab, triangles.T)
    return part.sum()


# speedup vs baseline: 9.9409x; 1.0752x over previous
import functools

import jax
import jax.numpy as jnp
import numpy as np
from jax import lax
from jax.experimental import pallas as pl
from jax.experimental.pallas import tpu as pltpu
from jax.experimental.pallas import tpu_sc as plsc

NC = 2
NS = 16
L = 16

B, C, H, W = 16, 3, 256, 256
T = 16384
D = 128
NW = NC * NS
TPW = T // NW
CH = 128
NCHUNK = TPW // CH
EPS2 = np.float32(1e-24)
INV_EPS = np.float32(1e12)
MEAN_SCALE = np.float32(1.0 / (B * 3 * T))


def _rsqrt(s):
    i = lax.bitcast_convert_type(s, jnp.int32)
    i = np.int32(0x5F3759DF) - (i >> 1)
    r = lax.bitcast_convert_type(i, jnp.float32)
    hs = s * np.float32(-0.5)
    r = r * (np.float32(1.5) + hs * r * r)
    r = r * (np.float32(1.5) + hs * r * r)
    return jnp.where(s < EPS2, INV_EPS, r)


_HB = 8


def _tc_prep_body(uo_ref, ug_ref, tab_ref):
    z = jnp.zeros((D - 2 * C * B, W), jnp.float32)
    for k in range(_HB):
        o = jnp.transpose(uo_ref[:, :, k, :], (1, 0, 2)).reshape(C * B, W)
        g = jnp.transpose(ug_ref[:, :, k, :], (1, 0, 2)).reshape(C * B, W)
        stacked = jnp.concatenate([o, g, z], axis=0)
        tab_ref[pl.ds(k * W, W), :] = stacked.T


_tc_prep = pl.pallas_call(
    _tc_prep_body,
    grid=(H // _HB,),
    in_specs=[
        pl.BlockSpec((B, C, _HB, W), lambda h: (0, 0, h, 0)),
        pl.BlockSpec((B, C, _HB, W), lambda h: (0, 0, h, 0)),
    ],
    out_specs=pl.BlockSpec((_HB * W, D), lambda h: (h, 0)),
    out_shape=jax.ShapeDtypeStruct((H * W, D), jnp.float32),
)


def _sc_body(tab, tri, out,
             tri_v, idx0, idx1, idx2,
             r0, r1, r2,
             accv, outv, shv, redv, sem):
    cid = lax.axis_index("c")
    sid = lax.axis_index("s")
    wid = sid * NC + cid

    acc = jnp.zeros((L,), jnp.float32)
    for chunk in range(NCHUNK):
        base = wid * TPW + chunk * CH
        pltpu.sync_copy(tri.at[:, pl.ds(base, CH)], tri_v)
        for g in range(CH // L):
            sl = pl.ds(g * L, L)
            idx0[sl] = (tri_v[0, sl] << 8) + tri_v[1, sl]
            idx1[sl] = (tri_v[2, sl] << 8) + tri_v[3, sl]
            idx2[sl] = (tri_v[4, sl] << 8) + tri_v[5, sl]
        cps = [
            pltpu.async_copy(tab.at[idx0], r0, sem),
            pltpu.async_copy(tab.at[idx1], r1, sem),
            pltpu.async_copy(tab.at[idx2], r2, sem),
        ]
        for cp in cps:
            cp.wait()

        def tri_body(i, a):
            e1x = r1[i, pl.ds(0, L)] - r0[i, pl.ds(0, L)]
            e1y = r1[i, pl.ds(16, L)] - r0[i, pl.ds(16, L)]
            e1z = r1[i, pl.ds(32, L)] - r0[i, pl.ds(32, L)]
            e2x = r2[i, pl.ds(0, L)] - r0[i, pl.ds(0, L)]
            e2y = r2[i, pl.ds(16, L)] - r0[i, pl.ds(16, L)]
            e2z = r2[i, pl.ds(32, L)] - r0[i, pl.ds(32, L)]
            e3x = r2[i, pl.ds(0, L)] - r1[i, pl.ds(0, L)]
            e3y = r2[i, pl.ds(16, L)] - r1[i, pl.ds(16, L)]
            e3z = r2[i, pl.ds(32, L)] - r1[i, pl.ds(32, L)]
            f1x = r1[i, pl.ds(48, L)] - r0[i, pl.ds(48, L)]
            f1y = r1[i, pl.ds(64, L)] - r0[i, pl.ds(64, L)]
            f1z = r1[i, pl.ds(80, L)] - r0[i, pl.ds(80, L)]
            f2x = r2[i, pl.ds(48, L)] - r0[i, pl.ds(48, L)]
            f2y = r2[i, pl.ds(64, L)] - r0[i, pl.ds(64, L)]
            f2z = r2[i, pl.ds(80, L)] - r0[i, pl.ds(80, L)]
            nx = f1y * f2z - f1z * f2y
            ny = f1z * f2x - f1x * f2z
            nz = f1x * f2y - f1y * f2x
            rn = _rsqrt(nx * nx + ny * ny + nz * nz)
            rr1 = _rsqrt(e1x * e1x + e1y * e1y + e1z * e1z)
            rr2 = _rsqrt(e2x * e2x + e2y * e2y + e2z * e2z)
            rr3 = _rsqrt(e3x * e3x + e3y * e3y + e3z * e3z)
            c1 = jnp.abs(e1x * nx + e1y * ny + e1z * nz) * rr1
            c2 = jnp.abs(e2x * nx + e2y * ny + e2z * nz) * rr2
            c3 = jnp.abs(e3x * nx + e3y * ny + e3z * nz) * rr3
            return a + (c1 + c2 + c3) * rn

        acc = lax.fori_loop(0, CH, tri_body, acc)

    accv[...] = acc * MEAN_SCALE
    pltpu.sync_copy(accv, shv.at[sid])
    plsc.subcore_barrier()

    @pl.when(sid == 0)
    def _():
        pltpu.sync_copy(shv, redv)
        tot = redv[0]
        for s in range(1, NS):
            tot = tot + redv[s]
        for k in range(D // L):
            outv[pl.ds(k * L, L)] = jnp.zeros((L,), jnp.float32)
        outv[pl.ds(0, L)] = tot
        pltpu.sync_copy(outv, out.at[cid])


_sc_loss = functools.partial(
    pl.kernel,
    out_type=jax.ShapeDtypeStruct((NC, D), jnp.float32),
    mesh=plsc.VectorSubcoreMesh(
        core_axis_name="c", subcore_axis_name="s",
        num_cores=NC, num_subcores=NS),
    compiler_params=pltpu.CompilerParams(
        use_tc_tiling_on_sc=False, needs_layout_passes=False),
    scratch_types=[
        pltpu.VMEM((6, CH), jnp.int32),
        pltpu.VMEM((CH,), jnp.int32),
        pltpu.VMEM((CH,), jnp.int32),
        pltpu.VMEM((CH,), jnp.int32),
        pltpu.VMEM((CH, D), jnp.float32),
        pltpu.VMEM((CH, D), jnp.float32),
        pltpu.VMEM((CH, D), jnp.float32),
        pltpu.VMEM((L,), jnp.float32),
        pltpu.VMEM((D,), jnp.float32),
        pltpu.VMEM_SHARED((NS, L), jnp.float32),
        pltpu.VMEM((NS, L), jnp.float32),
        pltpu.SemaphoreType.DMA,
    ],
)(_sc_body)


def kernel(uvm_gt, uvm_out, triangles):
    tab = _tc_prep(uvm_out, uvm_gt)
    part = _sc_loss(tab, triangles.T)
    return part.sum()


# speedup vs baseline: 10.8885x; 1.0953x over previous
import functools

import jax
import jax.numpy as jnp
import numpy as np
from jax import lax
from jax.experimental import pallas as pl
from jax.experimental.pallas import tpu as pltpu
from jax.experimental.pallas import tpu_sc as plsc

NC = 2
NS = 16
L = 16

B, C, H, W = 16, 3, 256, 256
T = 16384
D = 128
NW = NC * NS
TPW = T // NW
CH = 128
NCHUNK = TPW // CH
EPS2 = np.float32(1e-24)
INV_EPS = np.float32(1e12)
MEAN_SCALE = np.float32(1.0 / (B * 3 * T))


def _rsqrt(s):
    i = lax.bitcast_convert_type(s, jnp.int32)
    i = np.int32(0x5F3759DF) - (i >> 1)
    r = lax.bitcast_convert_type(i, jnp.float32)
    hs = s * np.float32(-0.5)
    r = r * (np.float32(1.5) + hs * r * r)
    r = r * (np.float32(1.5) + hs * r * r)
    return jnp.where(s < EPS2, INV_EPS, r)


_HB = 8


def _tc_prep_body(uo_ref, ug_ref, tab_ref):
    z = jnp.zeros((D - 2 * C * B, W), jnp.float32)
    for k in range(_HB):
        o = jnp.transpose(uo_ref[:, :, k, :], (1, 0, 2)).reshape(C * B, W)
        g = jnp.transpose(ug_ref[:, :, k, :], (1, 0, 2)).reshape(C * B, W)
        stacked = jnp.concatenate([o, g, z], axis=0)
        tab_ref[pl.ds(k * W, W), :] = stacked.T


_tc_prep = pl.pallas_call(
    _tc_prep_body,
    grid=(H // _HB,),
    in_specs=[
        pl.BlockSpec((B, C, _HB, W), lambda h: (0, 0, h, 0)),
        pl.BlockSpec((B, C, _HB, W), lambda h: (0, 0, h, 0)),
    ],
    out_specs=pl.BlockSpec((_HB * W, D), lambda h: (h, 0)),
    out_shape=jax.ShapeDtypeStruct((H * W, D), jnp.float32),
)


def _sc_body(tab, tri, out,
             tri_v,
             idx0a, idx1a, idx2a, idx0b, idx1b, idx2b,
             r0a, r1a, r2a, r0b, r1b, r2b,
             accv, outv, shv, redv, sem_a, sem_b):
    cid = lax.axis_index("c")
    sid = lax.axis_index("s")
    wid = sid * NC + cid
    bufs = [(idx0a, idx1a, idx2a, r0a, r1a, r2a, sem_a),
            (idx0b, idx1b, idx2b, r0b, r1b, r2b, sem_b)]

    def fire(chunk, idx0, idx1, idx2, r0, r1, r2, sem):
        base = wid * TPW + chunk * CH
        pltpu.sync_copy(tri.at[:, pl.ds(base, CH)], tri_v)
        for g in range(CH // L):
            sl = pl.ds(g * L, L)
            idx0[sl] = (tri_v[0, sl] << 8) + tri_v[1, sl]
            idx1[sl] = (tri_v[2, sl] << 8) + tri_v[3, sl]
            idx2[sl] = (tri_v[4, sl] << 8) + tri_v[5, sl]
        return [pltpu.async_copy(tab.at[idx0], r0, sem),
                pltpu.async_copy(tab.at[idx1], r1, sem),
                pltpu.async_copy(tab.at[idx2], r2, sem)]

    acc = jnp.zeros((L,), jnp.float32)
    cps = fire(0, *bufs[0])
    for chunk in range(NCHUNK):
        nxt = fire(chunk + 1, *bufs[(chunk + 1) % 2]) \
            if chunk + 1 < NCHUNK else None
        for cp in cps:
            cp.wait()
        cps = nxt
        _, _, _, r0, r1, r2, _ = bufs[chunk % 2]

        def tri_body(i, a):
            e1x = r1[i, pl.ds(0, L)] - r0[i, pl.ds(0, L)]
            e1y = r1[i, pl.ds(16, L)] - r0[i, pl.ds(16, L)]
            e1z = r1[i, pl.ds(32, L)] - r0[i, pl.ds(32, L)]
            e2x = r2[i, pl.ds(0, L)] - r0[i, pl.ds(0, L)]
            e2y = r2[i, pl.ds(16, L)] - r0[i, pl.ds(16, L)]
            e2z = r2[i, pl.ds(32, L)] - r0[i, pl.ds(32, L)]
            e3x = r2[i, pl.ds(0, L)] - r1[i, pl.ds(0, L)]
            e3y = r2[i, pl.ds(16, L)] - r1[i, pl.ds(16, L)]
            e3z = r2[i, pl.ds(32, L)] - r1[i, pl.ds(32, L)]
            f1x = r1[i, pl.ds(48, L)] - r0[i, pl.ds(48, L)]
            f1y = r1[i, pl.ds(64, L)] - r0[i, pl.ds(64, L)]
            f1z = r1[i, pl.ds(80, L)] - r0[i, pl.ds(80, L)]
            f2x = r2[i, pl.ds(48, L)] - r0[i, pl.ds(48, L)]
            f2y = r2[i, pl.ds(64, L)] - r0[i, pl.ds(64, L)]
            f2z = r2[i, pl.ds(80, L)] - r0[i, pl.ds(80, L)]
            nx = f1y * f2z - f1z * f2y
            ny = f1z * f2x - f1x * f2z
            nz = f1x * f2y - f1y * f2x
            rn = _rsqrt(nx * nx + ny * ny + nz * nz)
            rr1 = _rsqrt(e1x * e1x + e1y * e1y + e1z * e1z)
            rr2 = _rsqrt(e2x * e2x + e2y * e2y + e2z * e2z)
            rr3 = _rsqrt(e3x * e3x + e3y * e3y + e3z * e3z)
            c1 = jnp.abs(e1x * nx + e1y * ny + e1z * nz) * rr1
            c2 = jnp.abs(e2x * nx + e2y * ny + e2z * nz) * rr2
            c3 = jnp.abs(e3x * nx + e3y * ny + e3z * nz) * rr3
            return a + (c1 + c2 + c3) * rn

        acc = lax.fori_loop(0, CH, tri_body, acc)

    accv[...] = acc * MEAN_SCALE
    pltpu.sync_copy(accv, shv.at[sid])
    plsc.subcore_barrier()

    @pl.when(sid == 0)
    def _():
        pltpu.sync_copy(shv, redv)
        tot = redv[0]
        for s in range(1, NS):
            tot = tot + redv[s]
        for k in range(D // L):
            outv[pl.ds(k * L, L)] = jnp.zeros((L,), jnp.float32)
        outv[pl.ds(0, L)] = tot
        pltpu.sync_copy(outv, out.at[cid])


_sc_loss = functools.partial(
    pl.kernel,
    out_type=jax.ShapeDtypeStruct((NC, D), jnp.float32),
    mesh=plsc.VectorSubcoreMesh(
        core_axis_name="c", subcore_axis_name="s",
        num_cores=NC, num_subcores=NS),
    compiler_params=pltpu.CompilerParams(
        use_tc_tiling_on_sc=False, needs_layout_passes=False),
    scratch_types=[
        pltpu.VMEM((6, CH), jnp.int32),
        pltpu.VMEM((CH,), jnp.int32),
        pltpu.VMEM((CH,), jnp.int32),
        pltpu.VMEM((CH,), jnp.int32),
        pltpu.VMEM((CH,), jnp.int32),
        pltpu.VMEM((CH,), jnp.int32),
        pltpu.VMEM((CH,), jnp.int32),
        pltpu.VMEM((CH, D), jnp.float32),
        pltpu.VMEM((CH, D), jnp.float32),
        pltpu.VMEM((CH, D), jnp.float32),
        pltpu.VMEM((CH, D), jnp.float32),
        pltpu.VMEM((CH, D), jnp.float32),
        pltpu.VMEM((CH, D), jnp.float32),
        pltpu.VMEM((L,), jnp.float32),
        pltpu.VMEM((D,), jnp.float32),
        pltpu.VMEM_SHARED((NS, L), jnp.float32),
        pltpu.VMEM((NS, L), jnp.float32),
        pltpu.SemaphoreType.DMA,
        pltpu.SemaphoreType.DMA,
    ],
)(_sc_body)


def kernel(uvm_gt, uvm_out, triangles):
    tab = _tc_prep(uvm_out, uvm_gt)
    part = _sc_loss(tab, triangles.T)
    return part.sum()


# speedup vs baseline: 12.3041x; 1.1300x over previous
import functools

import jax
import jax.numpy as jnp
import numpy as np
from jax import lax
from jax.experimental import pallas as pl
from jax.experimental.pallas import tpu as pltpu
from jax.experimental.pallas import tpu_sc as plsc

NC = 2
NS = 16
L = 16

B, C, H, W = 16, 3, 256, 256
T = 16384
D = 128
NW = NC * NS
TPW = T // NW
CH = 128
NCHUNK = TPW // CH
EPS2 = np.float32(1e-24)
INV_EPS = np.float32(1e12)
MEAN_SCALE = np.float32(1.0 / (B * 3 * T))


def _rsqrt(s):
    i = lax.bitcast_convert_type(s, jnp.int32)
    i = np.int32(0x5F3759DF) - (i >> 1)
    r = lax.bitcast_convert_type(i, jnp.float32)
    hs = s * np.float32(-0.5)
    r = r * (np.float32(1.5) + hs * r * r)
    r = r * (np.float32(1.5) + hs * r * r)
    return jnp.where(s < EPS2, INV_EPS, r)


_HB = 16


def _tc_prep_body(uo_ref, ug_ref, tab_ref):
    z = jnp.zeros((D - 2 * C * B, W), jnp.float32)
    for k in range(_HB):
        o = jnp.transpose(uo_ref[:, :, k, :], (1, 0, 2)).reshape(C * B, W)
        g = jnp.transpose(ug_ref[:, :, k, :], (1, 0, 2)).reshape(C * B, W)
        stacked = jnp.concatenate([o, g, z], axis=0)
        tab_ref[pl.ds(k * W, W), :] = stacked.T


_tc_prep = pl.pallas_call(
    _tc_prep_body,
    grid=(H // _HB,),
    in_specs=[
        pl.BlockSpec((B, C, _HB, W), lambda h: (0, 0, h, 0)),
        pl.BlockSpec((B, C, _HB, W), lambda h: (0, 0, h, 0)),
    ],
    out_specs=pl.BlockSpec((_HB * W, D), lambda h: (h, 0)),
    out_shape=jax.ShapeDtypeStruct((H * W, D), jnp.float32),
)


def _sc_body(tab, tri, out,
             tri_v,
             idx0a, idx1a, idx2a, idx0b, idx1b, idx2b,
             r0a, r1a, r2a, r0b, r1b, r2b,
             accv, outv, shv, redv, sem_a, sem_b):
    cid = lax.axis_index("c")
    sid = lax.axis_index("s")
    wid = sid * NC + cid
    bufs = [(idx0a, idx1a, idx2a, r0a, r1a, r2a, sem_a),
            (idx0b, idx1b, idx2b, r0b, r1b, r2b, sem_b)]

    def fire(chunk, idx0, idx1, idx2, r0, r1, r2, sem):
        base = wid * TPW + chunk * CH
        pltpu.sync_copy(tri.at[:, pl.ds(base, CH)], tri_v)
        for g in range(CH // L):
            sl = pl.ds(g * L, L)
            idx0[sl] = (tri_v[0, sl] << 8) + tri_v[1, sl]
            idx1[sl] = (tri_v[2, sl] << 8) + tri_v[3, sl]
            idx2[sl] = (tri_v[4, sl] << 8) + tri_v[5, sl]
        return [pltpu.async_copy(tab.at[idx0], r0, sem),
                pltpu.async_copy(tab.at[idx1], r1, sem),
                pltpu.async_copy(tab.at[idx2], r2, sem)]

    acc = jnp.zeros((L,), jnp.float32)
    cps = fire(0, *bufs[0])
    for chunk in range(NCHUNK):
        nxt = fire(chunk + 1, *bufs[(chunk + 1) % 2]) \
            if chunk + 1 < NCHUNK else None
        for cp in cps:
            cp.wait()
        cps = nxt
        _, _, _, r0, r1, r2, _ = bufs[chunk % 2]

        def tri_body(i, a):
            e1x = r1[i, pl.ds(0, L)] - r0[i, pl.ds(0, L)]
            e1y = r1[i, pl.ds(16, L)] - r0[i, pl.ds(16, L)]
            e1z = r1[i, pl.ds(32, L)] - r0[i, pl.ds(32, L)]
            e2x = r2[i, pl.ds(0, L)] - r0[i, pl.ds(0, L)]
            e2y = r2[i, pl.ds(16, L)] - r0[i, pl.ds(16, L)]
            e2z = r2[i, pl.ds(32, L)] - r0[i, pl.ds(32, L)]
            e3x = r2[i, pl.ds(0, L)] - r1[i, pl.ds(0, L)]
            e3y = r2[i, pl.ds(16, L)] - r1[i, pl.ds(16, L)]
            e3z = r2[i, pl.ds(32, L)] - r1[i, pl.ds(32, L)]
            f1x = r1[i, pl.ds(48, L)] - r0[i, pl.ds(48, L)]
            f1y = r1[i, pl.ds(64, L)] - r0[i, pl.ds(64, L)]
            f1z = r1[i, pl.ds(80, L)] - r0[i, pl.ds(80, L)]
            f2x = r2[i, pl.ds(48, L)] - r0[i, pl.ds(48, L)]
            f2y = r2[i, pl.ds(64, L)] - r0[i, pl.ds(64, L)]
            f2z = r2[i, pl.ds(80, L)] - r0[i, pl.ds(80, L)]
            nx = f1y * f2z - f1z * f2y
            ny = f1z * f2x - f1x * f2z
            nz = f1x * f2y - f1y * f2x
            rn = _rsqrt(nx * nx + ny * ny + nz * nz)
            rr1 = _rsqrt(e1x * e1x + e1y * e1y + e1z * e1z)
            rr2 = _rsqrt(e2x * e2x + e2y * e2y + e2z * e2z)
            rr3 = _rsqrt(e3x * e3x + e3y * e3y + e3z * e3z)
            c1 = jnp.abs(e1x * nx + e1y * ny + e1z * nz) * rr1
            c2 = jnp.abs(e2x * nx + e2y * ny + e2z * nz) * rr2
            c3 = jnp.abs(e3x * nx + e3y * ny + e3z * nz) * rr3
            return a + (c1 + c2 + c3) * rn

        acc = lax.fori_loop(0, CH, tri_body, acc)

    accv[...] = acc * MEAN_SCALE
    pltpu.sync_copy(accv, shv.at[sid])
    plsc.subcore_barrier()

    @pl.when(sid == 0)
    def _():
        pltpu.sync_copy(shv, redv)
        tot = redv[0]
        for s in range(1, NS):
            tot = tot + redv[s]
        for k in range(D // L):
            outv[pl.ds(k * L, L)] = jnp.zeros((L,), jnp.float32)
        outv[pl.ds(0, L)] = tot
        pltpu.sync_copy(outv, out.at[cid])


_sc_loss = functools.partial(
    pl.kernel,
    out_type=jax.ShapeDtypeStruct((NC, D), jnp.float32),
    mesh=plsc.VectorSubcoreMesh(
        core_axis_name="c", subcore_axis_name="s",
        num_cores=NC, num_subcores=NS),
    compiler_params=pltpu.CompilerParams(
        use_tc_tiling_on_sc=False, needs_layout_passes=False),
    scratch_types=[
        pltpu.VMEM((6, CH), jnp.int32),
        pltpu.VMEM((CH,), jnp.int32),
        pltpu.VMEM((CH,), jnp.int32),
        pltpu.VMEM((CH,), jnp.int32),
        pltpu.VMEM((CH,), jnp.int32),
        pltpu.VMEM((CH,), jnp.int32),
        pltpu.VMEM((CH,), jnp.int32),
        pltpu.VMEM((CH, D), jnp.float32),
        pltpu.VMEM((CH, D), jnp.float32),
        pltpu.VMEM((CH, D), jnp.float32),
        pltpu.VMEM((CH, D), jnp.float32),
        pltpu.VMEM((CH, D), jnp.float32),
        pltpu.VMEM((CH, D), jnp.float32),
        pltpu.VMEM((L,), jnp.float32),
        pltpu.VMEM((D,), jnp.float32),
        pltpu.VMEM_SHARED((NS, L), jnp.float32),
        pltpu.VMEM((NS, L), jnp.float32),
        pltpu.SemaphoreType.DMA,
        pltpu.SemaphoreType.DMA,
    ],
)(_sc_body)


def kernel(uvm_gt, uvm_out, triangles):
    tab = _tc_prep(uvm_out, uvm_gt)
    part = _sc_loss(tab, triangles.T)
    return part.sum()


# speedup vs baseline: 13.0696x; 1.0622x over previous
import functools

import jax
import jax.numpy as jnp
import numpy as np
from jax import lax
from jax.experimental import pallas as pl
from jax.experimental.pallas import tpu as pltpu
from jax.experimental.pallas import tpu_sc as plsc

NC = 2
NS = 16
L = 16

B, C, H, W = 16, 3, 256, 256
T = 16384
D = 128
NW = NC * NS
TPW = T // NW
CH = 128
NCHUNK = TPW // CH
EPS2 = np.float32(1e-24)
INV_EPS = np.float32(1e12)
MEAN_SCALE = np.float32(1.0 / (B * 3 * T))


def _rsqrt(s):
    i = lax.bitcast_convert_type(s, jnp.int32)
    i = np.int32(0x5F3759DF) - (i >> 1)
    r = lax.bitcast_convert_type(i, jnp.float32)
    hs = s * np.float32(-0.5)
    r = r * (np.float32(1.5) + hs * r * r)
    r = r * (np.float32(1.5) + hs * r * r)
    return jnp.where(s < EPS2, INV_EPS, r)


_HB = 32


def _tc_prep_body(uo_ref, ug_ref, tab_ref):
    z = jnp.zeros((D - 2 * C * B, W), jnp.float32)
    for k in range(_HB):
        o = jnp.transpose(uo_ref[:, :, k, :], (1, 0, 2)).reshape(C * B, W)
        g = jnp.transpose(ug_ref[:, :, k, :], (1, 0, 2)).reshape(C * B, W)
        stacked = jnp.concatenate([o, g, z], axis=0)
        tab_ref[pl.ds(k * W, W), :] = stacked.T


_tc_prep = pl.pallas_call(
    _tc_prep_body,
    grid=(H // _HB,),
    in_specs=[
        pl.BlockSpec((B, C, _HB, W), lambda h: (0, 0, h, 0)),
        pl.BlockSpec((B, C, _HB, W), lambda h: (0, 0, h, 0)),
    ],
    out_specs=pl.BlockSpec((_HB * W, D), lambda h: (h, 0)),
    out_shape=jax.ShapeDtypeStruct((H * W, D), jnp.float32),
)


def _sc_body(tab, tri, out,
             tri_v,
             idx0a, idx1a, idx2a, idx0b, idx1b, idx2b,
             r0a, r1a, r2a, r0b, r1b, r2b,
             accv, outv, shv, redv, sem_a, sem_b):
    cid = lax.axis_index("c")
    sid = lax.axis_index("s")
    wid = sid * NC + cid
    bufs = [(idx0a, idx1a, idx2a, r0a, r1a, r2a, sem_a),
            (idx0b, idx1b, idx2b, r0b, r1b, r2b, sem_b)]

    def fire(chunk, idx0, idx1, idx2, r0, r1, r2, sem):
        base = wid * TPW + chunk * CH
        pltpu.sync_copy(tri.at[:, pl.ds(base, CH)], tri_v)
        for g in range(CH // L):
            sl = pl.ds(g * L, L)
            idx0[sl] = (tri_v[0, sl] << 8) + tri_v[1, sl]
            idx1[sl] = (tri_v[2, sl] << 8) + tri_v[3, sl]
            idx2[sl] = (tri_v[4, sl] << 8) + tri_v[5, sl]
        return [pltpu.async_copy(tab.at[idx0], r0, sem),
                pltpu.async_copy(tab.at[idx1], r1, sem),
                pltpu.async_copy(tab.at[idx2], r2, sem)]

    acc = jnp.zeros((L,), jnp.float32)
    cps = fire(0, *bufs[0])
    for chunk in range(NCHUNK):
        nxt = fire(chunk + 1, *bufs[(chunk + 1) % 2]) \
            if chunk + 1 < NCHUNK else None
        for cp in cps:
            cp.wait()
        cps = nxt
        _, _, _, r0, r1, r2, _ = bufs[chunk % 2]

        def tri_body(i, a):
            e1x = r1[i, pl.ds(0, L)] - r0[i, pl.ds(0, L)]
            e1y = r1[i, pl.ds(16, L)] - r0[i, pl.ds(16, L)]
            e1z = r1[i, pl.ds(32, L)] - r0[i, pl.ds(32, L)]
            e2x = r2[i, pl.ds(0, L)] - r0[i, pl.ds(0, L)]
            e2y = r2[i, pl.ds(16, L)] - r0[i, pl.ds(16, L)]
            e2z = r2[i, pl.ds(32, L)] - r0[i, pl.ds(32, L)]
            e3x = r2[i, pl.ds(0, L)] - r1[i, pl.ds(0, L)]
            e3y = r2[i, pl.ds(16, L)] - r1[i, pl.ds(16, L)]
            e3z = r2[i, pl.ds(32, L)] - r1[i, pl.ds(32, L)]
            f1x = r1[i, pl.ds(48, L)] - r0[i, pl.ds(48, L)]
            f1y = r1[i, pl.ds(64, L)] - r0[i, pl.ds(64, L)]
            f1z = r1[i, pl.ds(80, L)] - r0[i, pl.ds(80, L)]
            f2x = r2[i, pl.ds(48, L)] - r0[i, pl.ds(48, L)]
            f2y = r2[i, pl.ds(64, L)] - r0[i, pl.ds(64, L)]
            f2z = r2[i, pl.ds(80, L)] - r0[i, pl.ds(80, L)]
            nx = f1y * f2z - f1z * f2y
            ny = f1z * f2x - f1x * f2z
            nz = f1x * f2y - f1y * f2x
            rn = _rsqrt(nx * nx + ny * ny + nz * nz)
            rr1 = _rsqrt(e1x * e1x + e1y * e1y + e1z * e1z)
            rr2 = _rsqrt(e2x * e2x + e2y * e2y + e2z * e2z)
            rr3 = _rsqrt(e3x * e3x + e3y * e3y + e3z * e3z)
            c1 = jnp.abs(e1x * nx + e1y * ny + e1z * nz) * rr1
            c2 = jnp.abs(e2x * nx + e2y * ny + e2z * nz) * rr2
            c3 = jnp.abs(e3x * nx + e3y * ny + e3z * nz) * rr3
            return a + (c1 + c2 + c3) * rn

        acc = lax.fori_loop(0, CH, tri_body, acc)

    accv[...] = acc * MEAN_SCALE
    pltpu.sync_copy(accv, shv.at[sid])
    plsc.subcore_barrier()

    @pl.when(sid == 0)
    def _():
        pltpu.sync_copy(shv, redv)
        tot = redv[0]
        for s in range(1, NS):
            tot = tot + redv[s]
        for k in range(D // L):
            outv[pl.ds(k * L, L)] = jnp.zeros((L,), jnp.float32)
        outv[pl.ds(0, L)] = tot
        pltpu.sync_copy(outv, out.at[cid])


_sc_loss = functools.partial(
    pl.kernel,
    out_type=jax.ShapeDtypeStruct((NC, D), jnp.float32),
    mesh=plsc.VectorSubcoreMesh(
        core_axis_name="c", subcore_axis_name="s",
        num_cores=NC, num_subcores=NS),
    compiler_params=pltpu.CompilerParams(
        use_tc_tiling_on_sc=False, needs_layout_passes=False),
    scratch_types=[
        pltpu.VMEM((6, CH), jnp.int32),
        pltpu.VMEM((CH,), jnp.int32),
        pltpu.VMEM((CH,), jnp.int32),
        pltpu.VMEM((CH,), jnp.int32),
        pltpu.VMEM((CH,), jnp.int32),
        pltpu.VMEM((CH,), jnp.int32),
        pltpu.VMEM((CH,), jnp.int32),
        pltpu.VMEM((CH, D), jnp.float32),
        pltpu.VMEM((CH, D), jnp.float32),
        pltpu.VMEM((CH, D), jnp.float32),
        pltpu.VMEM((CH, D), jnp.float32),
        pltpu.VMEM((CH, D), jnp.float32),
        pltpu.VMEM((CH, D), jnp.float32),
        pltpu.VMEM((L,), jnp.float32),
        pltpu.VMEM((D,), jnp.float32),
        pltpu.VMEM_SHARED((NS, L), jnp.float32),
        pltpu.VMEM((NS, L), jnp.float32),
        pltpu.SemaphoreType.DMA,
        pltpu.SemaphoreType.DMA,
    ],
)(_sc_body)


def kernel(uvm_gt, uvm_out, triangles):
    tab = _tc_prep(uvm_out, uvm_gt)
    part = _sc_loss(tab, triangles.T)
    return part.sum()
